# baseline (speedup 1.0000x reference)
import dataclasses
import functools

import jax
import jax.numpy as jnp
from jax import lax
from jax.experimental import pallas as pl
from jax.experimental.pallas import tpu as pltpu
from jax.experimental.pallas import tpu_sc as plsc

NC = 2
NS = 16
L = 16
NW = NC * NS

SP = 10240
K = 128
NB = 16
ZR = 64


def _sc_gather_segsum(table, gidx, sidx):
  T, D = table.shape
  CHT = gidx.shape[0]
  per_w_ch = CHT // NW
  NBLK = per_w_ch // NB
  stripe = SP // NS

  mesh = plsc.VectorSubcoreMesh(
      core_axis_name="c", subcore_axis_name="s", num_cores=NC, num_subcores=NS
  )

  cp = pltpu.CompilerParams()
  if "needs_layout_passes" in pltpu.CompilerParams.__dataclass_fields__:
    cp = dataclasses.replace(cp, needs_layout_passes=False)

  @functools.partial(
      pl.kernel,
      compiler_params=cp,
      out_type=(
          jax.ShapeDtypeStruct((NC, SP, D), jnp.float32),
          jax.ShapeDtypeStruct((NW, SP), jnp.float32),
      ),
      mesh=mesh,
      scratch_types=[
          pltpu.VMEM((NB, K), jnp.int32),
          pltpu.VMEM((NB, K), jnp.int32),
          pltpu.VMEM((K, D), jnp.float32),
          pltpu.VMEM((K, D), jnp.float32),
          pltpu.VMEM((SP,), jnp.float32),
          pltpu.VMEM_SHARED((SP, D), jnp.float32),
          pltpu.SemaphoreType.DMA,
          pltpu.SemaphoreType.DMA,
          pltpu.SemaphoreType.DMA,
          pltpu.SemaphoreType.DMA,
      ],
  )
  def k(table_h, gidx_h, sidx_h, acc_h, cnt_h,
        gidx_v, sidx_v, rows0, rows1, cnt_v, acc_s,
        gsem0, gsem1, ssem0, ssem1):
    c = lax.axis_index("c")
    s = lax.axis_index("s")
    wid = s * NC + c

    rows = (rows0, rows1)
    gsem = (gsem0, gsem1)
    ssem = (ssem0, ssem1)

    zero16 = jnp.zeros((L,), jnp.float32)
    ones16 = jnp.ones((L,), jnp.float32)

    @pl.loop(0, K)
    def _(i):
      @pl.loop(0, D // L)
      def _(j):
        rows0[i, pl.ds(j * L, L)] = zero16

    @pl.loop(0, SP // L)
    def _(i):
      cnt_v[pl.ds(i * L, L)] = zero16

    row0 = s * stripe

    @pl.loop(0, stripe // K)
    def _(j):
      pltpu.sync_copy(rows0, acc_s.at[pl.ds(row0 + j * K, K)])

    plsc.subcore_barrier()

    base_ch = wid * per_w_ch

    @pl.loop(0, NBLK)
    def _(blk):
      ch0 = base_ch + blk * NB
      pltpu.sync_copy(gidx_h.at[pl.ds(ch0, NB)], gidx_v)
      pltpu.sync_copy(sidx_h.at[pl.ds(ch0, NB)], sidx_v)

      gd = [None] * NB
      sd = [None] * NB
      gd[0] = pltpu.async_copy(table_h.at[gidx_v.at[0]], rows[0], gsem[0])
      for j in range(NB):
        b = j % 2
        gd[j].wait()
        if j + 1 < NB:
          nb = (j + 1) % 2
          if j >= 1:
            sd[j - 1].wait()
          gd[j + 1] = pltpu.async_copy(
              table_h.at[gidx_v.at[j + 1]], rows[nb], gsem[nb])
        sd[j] = pltpu.async_copy(
            rows[b], acc_s.at[sidx_v.at[j]], ssem[b], add=True)

        for g in range(K // L):
          idx16 = sidx_v[j, pl.ds(g * L, L)]
          plsc.addupdate_scatter(cnt_v, [idx16], ones16)

      sd[NB - 2].wait()
      sd[NB - 1].wait()

    plsc.subcore_barrier()

    pltpu.sync_copy(acc_s.at[pl.ds(row0, stripe)],
                    acc_h.at[c].at[pl.ds(row0, stripe)])
    pltpu.sync_copy(cnt_v, cnt_h.at[wid])

  return k(table, gidx, sidx)


def _tc_norm(acc, cnt):
  _, sp, d = acc.shape
  B = 512

  def body(a_ref, c_ref, o_ref):
    n = jnp.sum(c_ref[...], axis=0)
    inv = 1.0 / jnp.maximum(n, 1.0)
    o_ref[...] = (a_ref[0] + a_ref[1]) * inv[:, None]

  return pl.pallas_call(
      body,
      grid=(sp // B,),
      in_specs=[
          pl.BlockSpec((NC, B, d), lambda i: (0, i, 0)),
          pl.BlockSpec((NW, B), lambda i: (0, i)),
      ],
      out_specs=pl.BlockSpec((B, d), lambda i: (i, 0)),
      out_shape=jax.ShapeDtypeStruct((sp, d), jnp.float32),
  )(acc, cnt)


def _tc_final(acc, cnt, dxdt_p, wc, bc):
  _, sp, d = acc.shape
  cdim = wc.shape[0]
  B = 512

  def body(a_ref, c_ref, dx_ref, w_ref, b_ref, o_ref):
    n = jnp.sum(c_ref[...], axis=0)
    inv = 1.0 / jnp.maximum(n, 1.0)
    agg = (a_ref[0] + a_ref[1]) * inv[:, None]
    out = jnp.zeros((B, d), jnp.float32)
    for cc in range(cdim):
      raw = lax.dot_general(
          agg, w_ref[cc], (((1,), (0,)), ((), ())),
          preferred_element_type=jnp.float32,
          precision=lax.Precision.HIGHEST,
      )
      raw = jnp.maximum(raw + b_ref[cc][None, :], 0.0)
      out = out + raw * dx_ref[:, cc][:, None]
    o_ref[...] = out

  return pl.pallas_call(
      body,
      grid=(sp // B,),
      in_specs=[
          pl.BlockSpec((NC, B, d), lambda i: (0, i, 0)),
          pl.BlockSpec((NW, B), lambda i: (0, i)),
          pl.BlockSpec((B, cdim), lambda i: (i, 0)),
          pl.BlockSpec((cdim, d, d), lambda i: (0, 0, 0)),
          pl.BlockSpec((cdim, d), lambda i: (0, 0)),
      ],
      out_specs=pl.BlockSpec((B, d), lambda i: (i, 0)),
      out_shape=jax.ShapeDtypeStruct((sp, d), jnp.float32),
  )(acc, cnt, dxdt_p, wc, bc)


def kernel(y, incidence, dxdt, W, b):
  n, d = y.shape
  cdim = dxdt.shape[1]
  nnz = incidence.shape[1]

  node_idx = incidence[0]
  edge_idx = incidence[1]

  chunk = NW * K * NB
  nnzp = ((nnz + chunk - 1) // chunk) * chunk
  pad = nnzp - nnz
  gpad = jnp.zeros((pad,), jnp.int32)
  spad = jnp.full((pad,), SP - 1, jnp.int32)
  node_g = jnp.concatenate([node_idx, gpad]).reshape(nnzp // K, K)
  edge_s = jnp.concatenate([edge_idx, spad]).reshape(nnzp // K, K)
  edge_g = jnp.concatenate([edge_idx, gpad]).reshape(nnzp // K, K)
  node_s = jnp.concatenate([node_idx, spad]).reshape(nnzp // K, K)

  acc1, cnt1 = _sc_gather_segsum(y, node_g, edge_s)
  edge_feat = _tc_norm(acc1, cnt1)

  acc2, cnt2 = _sc_gather_segsum(edge_feat, edge_g, node_s)

  wc = jnp.transpose(W.reshape(d, d, cdim), (2, 0, 1))
  bc = jnp.transpose(b.reshape(d, cdim), (1, 0))
  dxdt_p = jnp.concatenate(
      [dxdt, jnp.zeros((SP - n, cdim), jnp.float32)], axis=0
  )

  drift = _tc_final(acc2, cnt2, dxdt_p, wc, bc)
  return drift[:n]


# speedup vs baseline: 1.3511x; 1.3511x over previous
import dataclasses
import functools

import jax
import jax.numpy as jnp
import numpy as np
from jax import lax
from jax.experimental import pallas as pl
from jax.experimental.pallas import tpu as pltpu
from jax.experimental.pallas import tpu_sc as plsc

NC = 2
NS = 16
L = 16
NW = NC * NS

SP = 10240
K = 80
NB = 16


def _unpack_perm(d):
  m = np.arange(d).reshape(-1, 32)
  m = np.concatenate([m[:, 0::2], m[:, 1::2]], axis=1).reshape(-1)
  return m


def _sc_gather_segsum(table, gidx, sidx):
  spt, d2i = table.shape
  D = 2 * d2i
  CHT = gidx.shape[0]
  per_w_ch = CHT // NW
  NBLK = per_w_ch // NB
  stripe = SP // NS

  mesh = plsc.VectorSubcoreMesh(
      core_axis_name="c", subcore_axis_name="s", num_cores=NC, num_subcores=NS
  )

  cp = pltpu.CompilerParams(
      needs_layout_passes=False, use_tc_tiling_on_sc=False)

  @functools.partial(
      pl.kernel,
      compiler_params=cp,
      out_type=(
          jax.ShapeDtypeStruct((NC, SP, D), jnp.float32),
          jax.ShapeDtypeStruct((NW, SP), jnp.float32),
      ),
      mesh=mesh,
      scratch_types=[
          pltpu.VMEM((NB, K), jnp.int32),
          pltpu.VMEM((NB, K), jnp.int32),
          pltpu.VMEM((K, d2i), jnp.int32),
          pltpu.VMEM((K, d2i), jnp.int32),
          pltpu.VMEM((K, D), jnp.float32),
          pltpu.VMEM((K, D), jnp.float32),
          pltpu.VMEM((SP,), jnp.float32),
          pltpu.VMEM_SHARED((SP, D), jnp.float32),
          pltpu.SemaphoreType.DMA,
          pltpu.SemaphoreType.DMA,
          pltpu.SemaphoreType.DMA,
          pltpu.SemaphoreType.DMA,
      ],
  )
  def k(table_h, gidx_h, sidx_h, acc_h, cnt_h,
        gidx_v, sidx_v, bf0, bf1, rf0, rf1, cnt_v, acc_s,
        gsem0, gsem1, ssem0, ssem1):
    c = lax.axis_index("c")
    s = lax.axis_index("s")
    wid = s * NC + c

    bfs = (bf0, bf1)
    rfs = (rf0, rf1)
    gsem = (gsem0, gsem1)
    ssem = (ssem0, ssem1)

    zero16 = jnp.zeros((L,), jnp.float32)
    ones16 = jnp.ones((L,), jnp.float32)

    @pl.loop(0, SP // L)
    def _(i):
      cnt_v[pl.ds(i * L, L)] = zero16

    @pl.loop(0, K)
    def _(i):
      @pl.loop(0, D // L)
      def _(j):
        rf0[i, pl.ds(j * L, L)] = zero16

    row0 = s * stripe

    @pl.loop(0, stripe // K)
    def _(j):
      pltpu.sync_copy(rf0, acc_s.at[pl.ds(row0 + j * K, K)])

    plsc.subcore_barrier()

    def convert(src_i32, dst_f):
      @pl.loop(0, K)
      def _(i):
        for g in range(D // 32):
          v = plsc.bitcast(src_i32[i, pl.ds(g * L, L)], jnp.bfloat16)
          a, b = plsc.unpack(v, format=plsc.PackFormat.INTERLEAVED)
          dst_f[i, pl.ds(g * 32, L)] = a
          dst_f[i, pl.ds(g * 32 + L, L)] = b

    base_ch = wid * per_w_ch

    @pl.loop(0, NBLK)
    def _(blk):
      ch0 = base_ch + blk * NB
      pltpu.sync_copy(gidx_h.at[pl.ds(ch0, NB)], gidx_v)
      pltpu.sync_copy(sidx_h.at[pl.ds(ch0, NB)], sidx_v)

      gd = [None] * NB
      sd = [None] * NB
      gd[0] = pltpu.async_copy(table_h.at[gidx_v.at[0]], bfs[0], gsem[0])
      for j in range(NB):
        b = j % 2
        gd[j].wait()
        if j + 1 < NB:
          gd[j + 1] = pltpu.async_copy(
              table_h.at[gidx_v.at[j + 1]], bfs[1 - b], gsem[1 - b])
        if j >= 2:
          sd[j - 2].wait()
        convert(bfs[b], rfs[b])
        sd[j] = pltpu.async_copy(
            rfs[b], acc_s.at[sidx_v.at[j]], ssem[b], add=True)

        for g in range(K // L):
          idx16 = sidx_v[j, pl.ds(g * L, L)]
          plsc.addupdate_scatter(cnt_v, [idx16], ones16)

      sd[NB - 2].wait()
      sd[NB - 1].wait()

    plsc.subcore_barrier()

    pltpu.sync_copy(acc_s.at[pl.ds(row0, stripe)],
                    acc_h.at[c].at[pl.ds(row0, stripe)])
    pltpu.sync_copy(cnt_v, cnt_h.at[wid])

  return k(table, gidx, sidx)


def _tc_norm(acc, cnt):
  _, sp, d = acc.shape
  B = 512

  def body(a_ref, c_ref, o_ref):
    n = jnp.sum(c_ref[...], axis=0)
    inv = 1.0 / jnp.maximum(n, 1.0)
    o_ref[...] = ((a_ref[0] + a_ref[1]) * inv[:, None]).astype(jnp.bfloat16)

  return pl.pallas_call(
      body,
      grid=(sp // B,),
      in_specs=[
          pl.BlockSpec((NC, B, d), lambda i: (0, i, 0)),
          pl.BlockSpec((NW, B), lambda i: (0, i)),
      ],
      out_specs=pl.BlockSpec((B, d), lambda i: (i, 0)),
      out_shape=jax.ShapeDtypeStruct((sp, d), jnp.bfloat16),
  )(acc, cnt)


def _tc_final(acc, cnt, dxdt_p, wc, bc):
  _, sp, d = acc.shape
  cdim = wc.shape[0]
  B = 512

  def body(a_ref, c_ref, dx_ref, w_ref, b_ref, o_ref):
    n = jnp.sum(c_ref[...], axis=0)
    inv = 1.0 / jnp.maximum(n, 1.0)
    agg = (a_ref[0] + a_ref[1]) * inv[:, None]
    out = jnp.zeros((B, d), jnp.float32)
    for cc in range(cdim):
      raw = lax.dot_general(
          agg, w_ref[cc], (((1,), (0,)), ((), ())),
          preferred_element_type=jnp.float32,
          precision=lax.Precision.HIGHEST,
      )
      raw = jnp.maximum(raw + b_ref[cc][None, :], 0.0)
      out = out + raw * dx_ref[:, cc][:, None]
    o_ref[...] = out

  return pl.pallas_call(
      body,
      grid=(sp // B,),
      in_specs=[
          pl.BlockSpec((NC, B, d), lambda i: (0, i, 0)),
          pl.BlockSpec((NW, B), lambda i: (0, i)),
          pl.BlockSpec((B, cdim), lambda i: (i, 0)),
          pl.BlockSpec((cdim, d, d), lambda i: (0, 0, 0)),
          pl.BlockSpec((cdim, d), lambda i: (0, 0)),
      ],
      out_specs=pl.BlockSpec((B, d), lambda i: (i, 0)),
      out_shape=jax.ShapeDtypeStruct((sp, d), jnp.float32),
  )(acc, cnt, dxdt_p, wc, bc)


def kernel(y, incidence, dxdt, W, b):
  n, d = y.shape
  cdim = dxdt.shape[1]
  nnz = incidence.shape[1]

  node_idx = incidence[0]
  edge_idx = incidence[1]

  chunk = NW * K * NB
  nnzp = ((nnz + chunk - 1) // chunk) * chunk
  pad = nnzp - nnz
  gpad = jnp.zeros((pad,), jnp.int32)
  spad = jnp.full((pad,), SP - 1, jnp.int32)
  node_g = jnp.concatenate([node_idx, gpad]).reshape(nnzp // K, K)
  edge_s = jnp.concatenate([edge_idx, spad]).reshape(nnzp // K, K)
  edge_g = jnp.concatenate([edge_idx, gpad]).reshape(nnzp // K, K)
  node_s = jnp.concatenate([node_idx, spad]).reshape(nnzp // K, K)

  def _as_i32(x_bf16):
    sp_, d_ = x_bf16.shape
    return lax.bitcast_convert_type(
        x_bf16.reshape(sp_, d_ // 2, 2), jnp.int32)

  y_p = jnp.concatenate(
      [y, jnp.zeros((SP - n, d), jnp.float32)], axis=0
  ).astype(jnp.bfloat16)

  acc1, cnt1 = _sc_gather_segsum(_as_i32(y_p), node_g, edge_s)
  edge_feat = _tc_norm(acc1, cnt1)

  acc2, cnt2 = _sc_gather_segsum(_as_i32(edge_feat), edge_g, node_s)

  m = _unpack_perm(d)
  m2 = m[m]
  wc = jnp.transpose(W.reshape(d, d, cdim), (2, 0, 1))[:, m2, :]
  bc = jnp.transpose(b.reshape(d, cdim), (1, 0))
  dxdt_p = jnp.concatenate(
      [dxdt, jnp.zeros((SP - n, cdim), jnp.float32)], axis=0
  )

  drift = _tc_final(acc2, cnt2, dxdt_p, wc, bc)
  return drift[:n]


# speedup vs baseline: 1.3515x; 1.0002x over previous
import dataclasses
import functools

import jax
import jax.numpy as jnp
import numpy as np
from jax import lax
from jax.experimental import pallas as pl
from jax.experimental.pallas import tpu as pltpu
from jax.experimental.pallas import tpu_sc as plsc

NC = 2
NS = 16
L = 16
NW = NC * NS

SP = 10240
K = 80
NB = 16


def _unpack_perm(d):
  m = np.arange(d).reshape(-1, 32)
  m = np.concatenate([m[:, 0::2], m[:, 1::2]], axis=1).reshape(-1)
  return m


def _sc_gather_segsum(table, gidx, sidx):
  spt, d2i = table.shape
  D = 2 * d2i
  CHT = gidx.shape[0]
  per_w_ch = CHT // NW
  NBLK = per_w_ch // NB
  stripe = SP // NS

  mesh = plsc.VectorSubcoreMesh(
      core_axis_name="c", subcore_axis_name="s", num_cores=NC, num_subcores=NS
  )

  cp = pltpu.CompilerParams(
      needs_layout_passes=False, use_tc_tiling_on_sc=False)

  @functools.partial(
      pl.kernel,
      compiler_params=cp,
      out_type=(
          jax.ShapeDtypeStruct((NC, SP, D), jnp.float32),
          jax.ShapeDtypeStruct((NW, SP), jnp.float32),
      ),
      mesh=mesh,
      scratch_types=[
          pltpu.VMEM((NB, K), jnp.int32),
          pltpu.VMEM((NB, K), jnp.int32),
          pltpu.VMEM((K, d2i), jnp.int32),
          pltpu.VMEM((K, d2i), jnp.int32),
          pltpu.VMEM((K, D), jnp.float32),
          pltpu.VMEM((K, D), jnp.float32),
          pltpu.VMEM((SP,), jnp.float32),
          pltpu.VMEM_SHARED((SP, D), jnp.float32),
          pltpu.SemaphoreType.DMA,
          pltpu.SemaphoreType.DMA,
          pltpu.SemaphoreType.DMA,
          pltpu.SemaphoreType.DMA,
      ],
  )
  def k(table_h, gidx_h, sidx_h, acc_h, cnt_h,
        gidx_v, sidx_v, bf0, bf1, rf0, rf1, cnt_v, acc_s,
        gsem0, gsem1, ssem0, ssem1):
    c = lax.axis_index("c")
    s = lax.axis_index("s")
    wid = s * NC + c

    bfs = (bf0, bf1)
    rfs = (rf0, rf1)
    gsem = (gsem0, gsem1)
    ssem = (ssem0, ssem1)

    zero16 = jnp.zeros((L,), jnp.float32)
    ones16 = jnp.ones((L,), jnp.float32)

    @pl.loop(0, SP // L)
    def _(i):
      cnt_v[pl.ds(i * L, L)] = zero16

    @pl.loop(0, K)
    def _(i):
      @pl.loop(0, D // L)
      def _(j):
        rf0[i, pl.ds(j * L, L)] = zero16

    row0 = s * stripe

    @pl.loop(0, stripe // K)
    def _(j):
      pltpu.sync_copy(rf0, acc_s.at[pl.ds(row0 + j * K, K)])

    plsc.subcore_barrier()

    hi_mask = jnp.full((L,), -65536, jnp.int32)

    def convert(src_i32, dst_f):
      @pl.loop(0, K)
      def _(i):
        for g in range(D // 32):
          v = src_i32[i, pl.ds(g * L, L)]
          a = plsc.bitcast(v << 16, jnp.float32)
          b = plsc.bitcast(v & hi_mask, jnp.float32)
          dst_f[i, pl.ds(g * 32, L)] = a
          dst_f[i, pl.ds(g * 32 + L, L)] = b

    base_ch = wid * per_w_ch

    @pl.loop(0, NBLK)
    def _(blk):
      ch0 = base_ch + blk * NB
      pltpu.sync_copy(gidx_h.at[pl.ds(ch0, NB)], gidx_v)
      pltpu.sync_copy(sidx_h.at[pl.ds(ch0, NB)], sidx_v)

      gd = [None] * NB
      sd = [None] * NB
      gd[0] = pltpu.async_copy(table_h.at[gidx_v.at[0]], bfs[0], gsem[0])
      for j in range(NB):
        b = j % 2
        gd[j].wait()
        if j + 1 < NB:
          gd[j + 1] = pltpu.async_copy(
              table_h.at[gidx_v.at[j + 1]], bfs[1 - b], gsem[1 - b])
        if j >= 2:
          sd[j - 2].wait()
        convert(bfs[b], rfs[b])
        sd[j] = pltpu.async_copy(
            rfs[b], acc_s.at[sidx_v.at[j]], ssem[b], add=True)

        for g in range(K // L):
          idx16 = sidx_v[j, pl.ds(g * L, L)]
          plsc.addupdate_scatter(cnt_v, [idx16], ones16)

      sd[NB - 2].wait()
      sd[NB - 1].wait()

    plsc.subcore_barrier()

    pltpu.sync_copy(acc_s.at[pl.ds(row0, stripe)],
                    acc_h.at[c].at[pl.ds(row0, stripe)])
    pltpu.sync_copy(cnt_v, cnt_h.at[wid])

  return k(table, gidx, sidx)


def _tc_norm(acc, cnt):
  _, sp, d = acc.shape
  B = 512

  def body(a_ref, c_ref, o_ref):
    n = jnp.sum(c_ref[...], axis=0)
    inv = 1.0 / jnp.maximum(n, 1.0)
    o_ref[...] = ((a_ref[0] + a_ref[1]) * inv[:, None]).astype(jnp.bfloat16)

  return pl.pallas_call(
      body,
      grid=(sp // B,),
      in_specs=[
          pl.BlockSpec((NC, B, d), lambda i: (0, i, 0)),
          pl.BlockSpec((NW, B), lambda i: (0, i)),
      ],
      out_specs=pl.BlockSpec((B, d), lambda i: (i, 0)),
      out_shape=jax.ShapeDtypeStruct((sp, d), jnp.bfloat16),
  )(acc, cnt)


def _tc_final(acc, cnt, dxdt_p, wc, bc):
  _, sp, d = acc.shape
  cdim = wc.shape[0]
  B = 512

  def body(a_ref, c_ref, dx_ref, w_ref, b_ref, o_ref):
    n = jnp.sum(c_ref[...], axis=0)
    inv = 1.0 / jnp.maximum(n, 1.0)
    agg = (a_ref[0] + a_ref[1]) * inv[:, None]
    out = jnp.zeros((B, d), jnp.float32)
    for cc in range(cdim):
      raw = lax.dot_general(
          agg, w_ref[cc], (((1,), (0,)), ((), ())),
          preferred_element_type=jnp.float32,
          precision=lax.Precision.HIGHEST,
      )
      raw = jnp.maximum(raw + b_ref[cc][None, :], 0.0)
      out = out + raw * dx_ref[:, cc][:, None]
    o_ref[...] = out

  return pl.pallas_call(
      body,
      grid=(sp // B,),
      in_specs=[
          pl.BlockSpec((NC, B, d), lambda i: (0, i, 0)),
          pl.BlockSpec((NW, B), lambda i: (0, i)),
          pl.BlockSpec((B, cdim), lambda i: (i, 0)),
          pl.BlockSpec((cdim, d, d), lambda i: (0, 0, 0)),
          pl.BlockSpec((cdim, d), lambda i: (0, 0)),
      ],
      out_specs=pl.BlockSpec((B, d), lambda i: (i, 0)),
      out_shape=jax.ShapeDtypeStruct((sp, d), jnp.float32),
  )(acc, cnt, dxdt_p, wc, bc)


def kernel(y, incidence, dxdt, W, b):
  n, d = y.shape
  cdim = dxdt.shape[1]
  nnz = incidence.shape[1]

  node_idx = incidence[0]
  edge_idx = incidence[1]

  chunk = NW * K * NB
  nnzp = ((nnz + chunk - 1) // chunk) * chunk
  pad = nnzp - nnz
  gpad = jnp.zeros((pad,), jnp.int32)
  spad = jnp.full((pad,), SP - 1, jnp.int32)
  node_g = jnp.concatenate([node_idx, gpad]).reshape(nnzp // K, K)
  edge_s = jnp.concatenate([edge_idx, spad]).reshape(nnzp // K, K)
  edge_g = jnp.concatenate([edge_idx, gpad]).reshape(nnzp // K, K)
  node_s = jnp.concatenate([node_idx, spad]).reshape(nnzp // K, K)

  def _as_i32(x_bf16):
    sp_, d_ = x_bf16.shape
    return lax.bitcast_convert_type(
        x_bf16.reshape(sp_, d_ // 2, 2), jnp.int32)

  y_p = jnp.concatenate(
      [y, jnp.zeros((SP - n, d), jnp.float32)], axis=0
  ).astype(jnp.bfloat16)

  acc1, cnt1 = _sc_gather_segsum(_as_i32(y_p), node_g, edge_s)
  edge_feat = _tc_norm(acc1, cnt1)

  acc2, cnt2 = _sc_gather_segsum(_as_i32(edge_feat), edge_g, node_s)

  m = _unpack_perm(d)
  m2 = m[m]
  wc = jnp.transpose(W.reshape(d, d, cdim), (2, 0, 1))[:, m2, :]
  bc = jnp.transpose(b.reshape(d, cdim), (1, 0))
  dxdt_p = jnp.concatenate(
      [dxdt, jnp.zeros((SP - n, cdim), jnp.float32)], axis=0
  )

  drift = _tc_final(acc2, cnt2, dxdt_p, wc, bc)
  return drift[:n]


# speedup vs baseline: 1.4161x; 1.0479x over previous
import dataclasses
import functools

import jax
import jax.numpy as jnp
import numpy as np
from jax import lax
from jax.experimental import pallas as pl
from jax.experimental.pallas import tpu as pltpu
from jax.experimental.pallas import tpu_sc as plsc

NC = 2
NS = 16
L = 16
NW = NC * NS

SP = 10240
K = 80
NB = 16


def _unpack_perm(d):
  m = np.empty((d,), np.int64)
  half = d // 2
  for g in range(d // 32):
    for t in range(16):
      m[g * 32 + t] = g * 16 + t
      m[g * 32 + 16 + t] = half + g * 16 + t
  return m


def _pack_pairs_bits(u):
  half = u.shape[-1] // 2
  r = u + jnp.uint32(0x8000)
  lo = r[:, :half] >> 16
  hi = r[:, half:] & jnp.uint32(0xFFFF0000)
  return lax.bitcast_convert_type(lo | hi, jnp.int32)


def _sc_gather_segsum(table, gidx, sidx):
  spt, d2i = table.shape
  D = 2 * d2i
  CHT = gidx.shape[0]
  per_w_ch = CHT // NW
  NBLK = per_w_ch // NB
  stripe = SP // NS

  mesh = plsc.VectorSubcoreMesh(
      core_axis_name="c", subcore_axis_name="s", num_cores=NC, num_subcores=NS
  )

  cp = pltpu.CompilerParams(
      needs_layout_passes=False, use_tc_tiling_on_sc=False)

  @functools.partial(
      pl.kernel,
      compiler_params=cp,
      out_type=(
          jax.ShapeDtypeStruct((NC, SP, D), jnp.float32),
          jax.ShapeDtypeStruct((NW, SP), jnp.float32),
      ),
      mesh=mesh,
      scratch_types=[
          pltpu.VMEM((NB, K), jnp.int32),
          pltpu.VMEM((NB, K), jnp.int32),
          pltpu.VMEM((K, d2i), jnp.int32),
          pltpu.VMEM((K, d2i), jnp.int32),
          pltpu.VMEM((K, D), jnp.float32),
          pltpu.VMEM((K, D), jnp.float32),
          pltpu.VMEM((SP,), jnp.float32),
          pltpu.VMEM_SHARED((SP, D), jnp.float32),
          pltpu.SemaphoreType.DMA,
          pltpu.SemaphoreType.DMA,
          pltpu.SemaphoreType.DMA,
          pltpu.SemaphoreType.DMA,
      ],
  )
  def k(table_h, gidx_h, sidx_h, acc_h, cnt_h,
        gidx_v, sidx_v, bf0, bf1, rf0, rf1, cnt_v, acc_s,
        gsem0, gsem1, ssem0, ssem1):
    c = lax.axis_index("c")
    s = lax.axis_index("s")
    wid = s * NC + c

    bfs = (bf0, bf1)
    rfs = (rf0, rf1)
    gsem = (gsem0, gsem1)
    ssem = (ssem0, ssem1)

    zero16 = jnp.zeros((L,), jnp.float32)
    ones16 = jnp.ones((L,), jnp.float32)

    @pl.loop(0, SP // L)
    def _(i):
      cnt_v[pl.ds(i * L, L)] = zero16

    @pl.loop(0, K)
    def _(i):
      @pl.loop(0, D // L)
      def _(j):
        rf0[i, pl.ds(j * L, L)] = zero16

    row0 = s * stripe

    @pl.loop(0, stripe // K)
    def _(j):
      pltpu.sync_copy(rf0, acc_s.at[pl.ds(row0 + j * K, K)])

    plsc.subcore_barrier()

    hi_mask = jnp.full((L,), -65536, jnp.int32)

    def convert(src_i32, dst_f):
      @pl.loop(0, K)
      def _(i):
        for g in range(D // 32):
          v = src_i32[i, pl.ds(g * L, L)]
          a = plsc.bitcast(v << 16, jnp.float32)
          b = plsc.bitcast(v & hi_mask, jnp.float32)
          dst_f[i, pl.ds(g * 32, L)] = a
          dst_f[i, pl.ds(g * 32 + L, L)] = b

    base_ch = wid * per_w_ch

    @pl.loop(0, NBLK)
    def _(blk):
      ch0 = base_ch + blk * NB
      pltpu.sync_copy(gidx_h.at[pl.ds(ch0, NB)], gidx_v)
      pltpu.sync_copy(sidx_h.at[pl.ds(ch0, NB)], sidx_v)

      gd = [None] * NB
      sd = [None] * NB
      gd[0] = pltpu.async_copy(table_h.at[gidx_v.at[0]], bfs[0], gsem[0])
      for j in range(NB):
        b = j % 2
        gd[j].wait()
        if j + 1 < NB:
          gd[j + 1] = pltpu.async_copy(
              table_h.at[gidx_v.at[j + 1]], bfs[1 - b], gsem[1 - b])
        if j >= 2:
          sd[j - 2].wait()
        convert(bfs[b], rfs[b])
        sd[j] = pltpu.async_copy(
            rfs[b], acc_s.at[sidx_v.at[j]], ssem[b], add=True)

        for g in range(K // L):
          idx16 = sidx_v[j, pl.ds(g * L, L)]
          plsc.addupdate_scatter(cnt_v, [idx16], ones16)

      sd[NB - 2].wait()
      sd[NB - 1].wait()

    plsc.subcore_barrier()

    pltpu.sync_copy(acc_s.at[pl.ds(row0, stripe)],
                    acc_h.at[c].at[pl.ds(row0, stripe)])
    pltpu.sync_copy(cnt_v, cnt_h.at[wid])

  return k(table, gidx, sidx)


def _tc_norm(acc, cnt):
  _, sp, d = acc.shape
  B = 512

  def body(a_ref, c_ref, o_ref):
    n = jnp.sum(c_ref[...], axis=0)
    inv = 1.0 / jnp.maximum(n, 1.0)
    feat = (a_ref[0] + a_ref[1]) * inv[:, None]
    o_ref[...] = _pack_pairs_bits(
        lax.bitcast_convert_type(feat, jnp.uint32))

  return pl.pallas_call(
      body,
      grid=(sp // B,),
      in_specs=[
          pl.BlockSpec((NC, B, d), lambda i: (0, i, 0)),
          pl.BlockSpec((NW, B), lambda i: (0, i)),
      ],
      out_specs=pl.BlockSpec((B, d // 2), lambda i: (i, 0)),
      out_shape=jax.ShapeDtypeStruct((sp, d // 2), jnp.int32),
  )(acc, cnt)


def _tc_final(acc, cnt, dxdt_p, wc, bc):
  _, sp, d = acc.shape
  cdim = wc.shape[0]
  B = 512

  def body(a_ref, c_ref, dx_ref, w_ref, b_ref, o_ref):
    n = jnp.sum(c_ref[...], axis=0)
    inv = 1.0 / jnp.maximum(n, 1.0)
    agg = (a_ref[0] + a_ref[1]) * inv[:, None]
    out = jnp.zeros((B, d), jnp.float32)
    for cc in range(cdim):
      raw = lax.dot_general(
          agg, w_ref[cc], (((1,), (0,)), ((), ())),
          preferred_element_type=jnp.float32,
          precision=lax.Precision.HIGHEST,
      )
      raw = jnp.maximum(raw + b_ref[cc][None, :], 0.0)
      out = out + raw * dx_ref[:, cc][:, None]
    o_ref[...] = out

  return pl.pallas_call(
      body,
      grid=(sp // B,),
      in_specs=[
          pl.BlockSpec((NC, B, d), lambda i: (0, i, 0)),
          pl.BlockSpec((NW, B), lambda i: (0, i)),
          pl.BlockSpec((B, cdim), lambda i: (i, 0)),
          pl.BlockSpec((cdim, d, d), lambda i: (0, 0, 0)),
          pl.BlockSpec((cdim, d), lambda i: (0, 0)),
      ],
      out_specs=pl.BlockSpec((B, d), lambda i: (i, 0)),
      out_shape=jax.ShapeDtypeStruct((sp, d), jnp.float32),
  )(acc, cnt, dxdt_p, wc, bc)


def kernel(y, incidence, dxdt, W, b):
  n, d = y.shape
  cdim = dxdt.shape[1]
  nnz = incidence.shape[1]

  node_idx = incidence[0]
  edge_idx = incidence[1]

  chunk = NW * K * NB
  nnzp = ((nnz + chunk - 1) // chunk) * chunk
  pad = nnzp - nnz
  gpad = jnp.zeros((pad,), jnp.int32)
  spad = jnp.full((pad,), SP - 1, jnp.int32)
  node_g = jnp.concatenate([node_idx, gpad]).reshape(nnzp // K, K)
  edge_s = jnp.concatenate([edge_idx, spad]).reshape(nnzp // K, K)
  edge_g = jnp.concatenate([edge_idx, gpad]).reshape(nnzp // K, K)
  node_s = jnp.concatenate([node_idx, spad]).reshape(nnzp // K, K)

  y_p = jnp.concatenate(
      [y, jnp.zeros((SP - n, d), jnp.float32)], axis=0
  )
  y_packed = _pack_pairs_bits(lax.bitcast_convert_type(y_p, jnp.uint32))

  acc1, cnt1 = _sc_gather_segsum(y_packed, node_g, edge_s)
  edge_feat = _tc_norm(acc1, cnt1)

  acc2, cnt2 = _sc_gather_segsum(edge_feat, edge_g, node_s)

  m = _unpack_perm(d)
  m2 = m[m]
  wc = jnp.transpose(W.reshape(d, d, cdim), (2, 0, 1))[:, m2, :]
  bc = jnp.transpose(b.reshape(d, cdim), (1, 0))
  dxdt_p = jnp.concatenate(
      [dxdt, jnp.zeros((SP - n, cdim), jnp.float32)], axis=0
  )

  drift = _tc_final(acc2, cnt2, dxdt_p, wc, bc)
  return drift[:n]


# speedup vs baseline: 1.5661x; 1.1059x over previous
import dataclasses
import functools

import jax
import jax.numpy as jnp
import numpy as np
from jax import lax
from jax.experimental import pallas as pl
from jax.experimental.pallas import tpu as pltpu
from jax.experimental.pallas import tpu_sc as plsc

NC = 2
NS = 16
L = 16
NW = NC * NS

SP = 10240
K = 80
NB = 16


def _unpack_perm(d):
  m = np.empty((d,), np.int64)
  half = d // 2
  for g in range(d // 32):
    for t in range(16):
      m[g * 32 + t] = g * 16 + t
      m[g * 32 + 16 + t] = half + g * 16 + t
  return m


def _pack_pairs_bits(u):
  half = u.shape[-1] // 2
  r = u + jnp.uint32(0x8000)
  lo = r[:, :half] >> 16
  hi = r[:, half:] & jnp.uint32(0xFFFF0000)
  return lax.bitcast_convert_type(lo | hi, jnp.int32)


def _sc_gather_segsum(table, gidx, sidx):
  spt, d2i = table.shape
  D = 2 * d2i
  CHT = gidx.shape[0]
  per_w_ch = CHT // NW
  NBLK = per_w_ch // NB
  stripe = SP // NS

  mesh = plsc.VectorSubcoreMesh(
      core_axis_name="c", subcore_axis_name="s", num_cores=NC, num_subcores=NS
  )

  cp = pltpu.CompilerParams(
      needs_layout_passes=False, use_tc_tiling_on_sc=False)

  @functools.partial(
      pl.kernel,
      compiler_params=cp,
      out_type=(
          jax.ShapeDtypeStruct((NC, SP, D), jnp.float32),
          jax.ShapeDtypeStruct((NW, SP), jnp.float32),
      ),
      mesh=mesh,
      scratch_types=[
          pltpu.VMEM((NB, K), jnp.int32),
          pltpu.VMEM((NB, K), jnp.int32),
          pltpu.VMEM((K, d2i), jnp.int32),
          pltpu.VMEM((K, d2i), jnp.int32),
          pltpu.VMEM((K, d2i), jnp.int32),
          pltpu.VMEM((K, D), jnp.float32),
          pltpu.VMEM((K, D), jnp.float32),
          pltpu.VMEM((SP,), jnp.float32),
          pltpu.VMEM_SHARED((SP, D), jnp.float32),
          pltpu.SemaphoreType.DMA,
          pltpu.SemaphoreType.DMA,
          pltpu.SemaphoreType.DMA,
          pltpu.SemaphoreType.DMA,
          pltpu.SemaphoreType.DMA,
      ],
  )
  def k(table_h, gidx_h, sidx_h, acc_h, cnt_h,
        gidx_v, sidx_v, bf0, bf1, bf2, rf0, rf1, cnt_v, acc_s,
        gsem0, gsem1, gsem2, ssem0, ssem1):
    c = lax.axis_index("c")
    s = lax.axis_index("s")
    wid = s * NC + c

    bfs = (bf0, bf1, bf2)
    rfs = (rf0, rf1)
    gsem = (gsem0, gsem1, gsem2)
    ssem = (ssem0, ssem1)

    zero16 = jnp.zeros((L,), jnp.float32)
    ones16 = jnp.ones((L,), jnp.float32)

    @pl.loop(0, SP // L)
    def _(i):
      cnt_v[pl.ds(i * L, L)] = zero16

    @pl.loop(0, K)
    def _(i):
      @pl.loop(0, D // L)
      def _(j):
        rf0[i, pl.ds(j * L, L)] = zero16

    row0 = s * stripe

    @pl.loop(0, stripe // K)
    def _(j):
      pltpu.sync_copy(rf0, acc_s.at[pl.ds(row0 + j * K, K)])

    plsc.subcore_barrier()

    hi_mask = jnp.full((L,), -65536, jnp.int32)

    def convert(src_i32, dst_f):
      @pl.loop(0, K)
      def _(i):
        for g in range(D // 32):
          v = src_i32[i, pl.ds(g * L, L)]
          a = plsc.bitcast(v << 16, jnp.float32)
          b = plsc.bitcast(v & hi_mask, jnp.float32)
          dst_f[i, pl.ds(g * 32, L)] = a
          dst_f[i, pl.ds(g * 32 + L, L)] = b

    base_ch = wid * per_w_ch

    @pl.loop(0, NBLK)
    def _(blk):
      ch0 = base_ch + blk * NB
      pltpu.sync_copy(gidx_h.at[pl.ds(ch0, NB)], gidx_v)
      pltpu.sync_copy(sidx_h.at[pl.ds(ch0, NB)], sidx_v)

      gd = [None] * NB
      sd = [None] * NB
      gd[0] = pltpu.async_copy(table_h.at[gidx_v.at[0]], bfs[0], gsem[0])
      gd[1] = pltpu.async_copy(table_h.at[gidx_v.at[1]], bfs[1], gsem[1])
      for j in range(NB):
        b3 = j % 3
        b = j % 2
        gd[j].wait()
        if j + 2 < NB:
          gd[j + 2] = pltpu.async_copy(
              table_h.at[gidx_v.at[j + 2]], bfs[(j + 2) % 3],
              gsem[(j + 2) % 3])
        if j >= 2:
          sd[j - 2].wait()
        convert(bfs[b3], rfs[b])
        sd[j] = pltpu.async_copy(
            rfs[b], acc_s.at[sidx_v.at[j]], ssem[b], add=True)

        for g in range(K // L):
          idx16 = sidx_v[j, pl.ds(g * L, L)]
          plsc.addupdate_scatter(cnt_v, [idx16], ones16)

      sd[NB - 2].wait()
      sd[NB - 1].wait()

    plsc.subcore_barrier()

    pltpu.sync_copy(acc_s.at[pl.ds(row0, stripe)],
                    acc_h.at[c].at[pl.ds(row0, stripe)])
    pltpu.sync_copy(cnt_v, cnt_h.at[wid])

  return k(table, gidx, sidx)


def _tc_norm(acc, cnt):
  _, sp, d = acc.shape
  B = 512

  def body(a_ref, c_ref, o_ref):
    n = jnp.sum(c_ref[...], axis=0)
    inv = 1.0 / jnp.maximum(n, 1.0)
    feat = (a_ref[0] + a_ref[1]) * inv[:, None]
    o_ref[...] = _pack_pairs_bits(
        lax.bitcast_convert_type(feat, jnp.uint32))

  return pl.pallas_call(
      body,
      grid=(sp // B,),
      in_specs=[
          pl.BlockSpec((NC, B, d), lambda i: (0, i, 0)),
          pl.BlockSpec((NW, B), lambda i: (0, i)),
      ],
      out_specs=pl.BlockSpec((B, d // 2), lambda i: (i, 0)),
      out_shape=jax.ShapeDtypeStruct((sp, d // 2), jnp.int32),
  )(acc, cnt)


def _tc_final(acc, cnt, dxdt_p, wc, bc):
  _, sp, d = acc.shape
  cdim = wc.shape[0]
  B = 512

  def body(a_ref, c_ref, dx_ref, w_ref, b_ref, o_ref):
    n = jnp.sum(c_ref[...], axis=0)
    inv = 1.0 / jnp.maximum(n, 1.0)
    agg = (a_ref[0] + a_ref[1]) * inv[:, None]
    out = jnp.zeros((B, d), jnp.float32)
    for cc in range(cdim):
      raw = lax.dot_general(
          agg, w_ref[cc], (((1,), (0,)), ((), ())),
          preferred_element_type=jnp.float32,
          precision=lax.Precision.HIGHEST,
      )
      raw = jnp.maximum(raw + b_ref[cc][None, :], 0.0)
      out = out + raw * dx_ref[:, cc][:, None]
    o_ref[...] = out

  return pl.pallas_call(
      body,
      grid=(sp // B,),
      in_specs=[
          pl.BlockSpec((NC, B, d), lambda i: (0, i, 0)),
          pl.BlockSpec((NW, B), lambda i: (0, i)),
          pl.BlockSpec((B, cdim), lambda i: (i, 0)),
          pl.BlockSpec((cdim, d, d), lambda i: (0, 0, 0)),
          pl.BlockSpec((cdim, d), lambda i: (0, 0)),
      ],
      out_specs=pl.BlockSpec((B, d), lambda i: (i, 0)),
      out_shape=jax.ShapeDtypeStruct((sp, d), jnp.float32),
  )(acc, cnt, dxdt_p, wc, bc)


def kernel(y, incidence, dxdt, W, b):
  n, d = y.shape
  cdim = dxdt.shape[1]
  nnz = incidence.shape[1]

  node_idx = incidence[0]
  edge_idx = incidence[1]

  chunk = NW * K * NB
  nnzp = ((nnz + chunk - 1) // chunk) * chunk
  pad = nnzp - nnz
  gpad = jnp.zeros((pad,), jnp.int32)
  spad = jnp.full((pad,), SP - 1, jnp.int32)
  node_g = jnp.concatenate([node_idx, gpad]).reshape(nnzp // K, K)
  edge_s = jnp.concatenate([edge_idx, spad]).reshape(nnzp // K, K)
  edge_g = jnp.concatenate([edge_idx, gpad]).reshape(nnzp // K, K)
  node_s = jnp.concatenate([node_idx, spad]).reshape(nnzp // K, K)

  y_p = jnp.concatenate(
      [y, jnp.zeros((SP - n, d), jnp.float32)], axis=0
  )
  y_packed = _pack_pairs_bits(lax.bitcast_convert_type(y_p, jnp.uint32))

  acc1, cnt1 = _sc_gather_segsum(y_packed, node_g, edge_s)
  edge_feat = _tc_norm(acc1, cnt1)

  acc2, cnt2 = _sc_gather_segsum(edge_feat, edge_g, node_s)

  m = _unpack_perm(d)
  m2 = m[m]
  wc = jnp.transpose(W.reshape(d, d, cdim), (2, 0, 1))[:, m2, :]
  bc = jnp.transpose(b.reshape(d, cdim), (1, 0))
  dxdt_p = jnp.concatenate(
      [dxdt, jnp.zeros((SP - n, cdim), jnp.float32)], axis=0
  )

  drift = _tc_final(acc2, cnt2, dxdt_p, wc, bc)
  return drift[:n]
